# Initial kernel scaffold; baseline (speedup 1.0000x reference)
import jax
import jax.numpy as jnp
from jax.experimental import pallas as pl


def kernel(x):
    raise NotImplementedError("write your pallas kernel here")


# speedup vs baseline: 1.0068x; 1.0068x over previous
import functools

import jax
import jax.numpy as jnp
import numpy as np
from jax.experimental import pallas as pl
from jax.experimental.pallas import tpu as pltpu


def _pooling_matrix(ho: int, wo: int, k: int) -> np.ndarray:
    wc = wo * k
    f = np.arange(ho * k * wc)
    col = (f // (k * wc)) * wo + (f % wc) // k
    q = np.zeros((f.size, ho * wo), np.float32)
    q[f, col] = 1.0
    return q


def _pool_kernel(q_ref, x_ref, o_ref, *, inv_vol):
    s = jnp.sum(x_ref[...], axis=1)
    y = jax.lax.dot_general(s.astype(jnp.bfloat16), q_ref[...],
                            (((1,), (0,)), ((), ())),
                            preferred_element_type=jnp.float32)
    o_ref[...] = y * inv_vol


def kernel(x):
    n, c, d, h, w = x.shape
    k = 8
    do, ho, wo = d // k, h // k, w // k
    hc, wc = ho * k, wo * k
    if (d, h, w) != (do * k, hc, wc):
        x = x[:, :, :do * k, :hc, :wc]

    G, F, Fo = n * c * do, hc * wc, ho * wo

    tb = min(256, G)
    if tb < G:
        tb = max(8, (tb // 8) * 8)
    num_i = pl.cdiv(G, tb)

    q = jnp.asarray(_pooling_matrix(ho, wo, k), dtype=jnp.bfloat16)

    slab_bytes = k * F * 4
    vmem_need = 2 * tb * slab_bytes + 2 * F * max(Fo, 128) * 2 \
        + 2 * tb * max(Fo, 128) * 4 + tb * F * 8
    vmem_limit = int(min(vmem_need + 4 * 1024 * 1024, 110 * 1024 * 1024))

    cost = pl.CostEstimate(
        flops=int(G * k * F + 2 * G * F * Fo),
        transcendentals=0,
        bytes_accessed=int(G * k * F * 4 + G * Fo * 4 + q.size * 2))

    out = pl.pallas_call(
        functools.partial(_pool_kernel, inv_vol=1.0 / float(k ** 3)),
        out_shape=jax.ShapeDtypeStruct((G, Fo), x.dtype),
        grid=(num_i,),
        in_specs=[
            pl.BlockSpec((F, Fo), lambda i: (0, 0)),
            pl.BlockSpec((tb, k, F), lambda i: (i, 0, 0)),
        ],
        out_specs=pl.BlockSpec((tb, Fo), lambda i: (i, 0)),
        compiler_params=pltpu.CompilerParams(
            dimension_semantics=("parallel",),
            vmem_limit_bytes=vmem_limit),
        cost_estimate=cost,
    )(q, x.reshape(G, k, F))

    return out.reshape(n, c, do, ho, wo)


# speedup vs baseline: 2.7539x; 2.7352x over previous
import functools

import jax
import jax.numpy as jnp
import numpy as np
from jax.experimental import pallas as pl
from jax.experimental.pallas import tpu as pltpu


def _w_pool_matrix(w: int, k: int) -> np.ndarray:
    wo = w // k
    q = np.zeros((w, wo), np.float32)
    q[np.arange(w), np.arange(w) // k] = 1.0
    return q


def _pool_kernel(qw_ref, x_ref, o_ref, *, k, inv_vol):
    tb, kk, h, w = x_ref.shape
    wo = qw_ref.shape[1]
    s = jnp.sum(x_ref[...], axis=1)
    s2 = s.reshape(tb * h, w)
    r = jax.lax.dot_general(s2.astype(jnp.bfloat16), qw_ref[...],
                            (((1,), (0,)), ((), ())),
                            preferred_element_type=jnp.float32)
    r4 = r.reshape(tb, h // k, k, wo)
    o_ref[...] = jnp.sum(r4, axis=2) * inv_vol


def kernel(x):
    n, c, d, h, w = x.shape
    k = 8
    do, ho, wo = d // k, h // k, w // k
    if (d, h, w) != (do * k, ho * k, wo * k):
        x = x[:, :, :do * k, :ho * k, :wo * k]
        h, w = ho * k, wo * k

    G = n * c * do
    x4 = x.reshape(G, k, h, w)

    tb = min(64, G)
    if tb < G:
        tb = max(8, (tb // 8) * 8)
    num_i = pl.cdiv(G, tb)

    qw = jnp.asarray(_w_pool_matrix(w, k), dtype=jnp.bfloat16)

    lanes = max(w, 128)
    block_vmem = tb * k * h * lanes * 4
    vmem_need = 2 * block_vmem + 2 * tb * ho * max(wo, 128) * 4 \
        + tb * h * lanes * 8 + 2 * h * max(wo, 128) * 2
    vmem_limit = int(min(vmem_need + 4 * 1024 * 1024, 110 * 1024 * 1024))

    cost = pl.CostEstimate(
        flops=int(G * k * h * w + 2 * G * h * w * wo + G * h * wo),
        transcendentals=0,
        bytes_accessed=int(G * k * h * w * 4 + G * ho * wo * 4 + qw.size * 2))

    out = pl.pallas_call(
        functools.partial(_pool_kernel, k=k, inv_vol=1.0 / float(k ** 3)),
        out_shape=jax.ShapeDtypeStruct((G, ho, wo), x.dtype),
        grid=(num_i,),
        in_specs=[
            pl.BlockSpec((w, wo), lambda i: (0, 0)),
            pl.BlockSpec((tb, k, h, w), lambda i: (i, 0, 0, 0)),
        ],
        out_specs=pl.BlockSpec((tb, ho, wo), lambda i: (i, 0, 0)),
        compiler_params=pltpu.CompilerParams(
            dimension_semantics=("parallel",),
            vmem_limit_bytes=vmem_limit),
        cost_estimate=cost,
    )(qw, x4)

    return out.reshape(n, c, do, ho, wo)


# speedup vs baseline: 2.7551x; 1.0005x over previous
import functools

import jax
import jax.numpy as jnp
import numpy as np
from jax.experimental import pallas as pl
from jax.experimental.pallas import tpu as pltpu


def _w_pool_matrix(w: int, k: int) -> np.ndarray:
    wo = w // k
    q = np.zeros((w, wo), np.float32)
    q[np.arange(w), np.arange(w) // k] = 1.0
    return q


def _pool_kernel(qw_ref, x_ref, o_ref, *, k, inv_vol):
    tb, kk, h, w = x_ref.shape
    wo = qw_ref.shape[1]
    s = jnp.sum(x_ref[...], axis=1)
    s2 = s.reshape(tb * h, w)
    r = jax.lax.dot_general(s2.astype(jnp.bfloat16), qw_ref[...],
                            (((1,), (0,)), ((), ())),
                            preferred_element_type=jnp.float32)
    r4 = r.reshape(tb, h // k, k, wo)
    o_ref[...] = jnp.sum(r4, axis=2) * inv_vol


def kernel(x):
    n, c, d, h, w = x.shape
    k = 8
    do, ho, wo = d // k, h // k, w // k
    if (d, h, w) != (do * k, ho * k, wo * k):
        x = x[:, :, :do * k, :ho * k, :wo * k]
        h, w = ho * k, wo * k

    G = n * c * do
    x4 = x.reshape(G, k, h, w)

    tb = min(128, G)
    if tb < G:
        tb = max(8, (tb // 8) * 8)
    num_i = pl.cdiv(G, tb)

    qw = jnp.asarray(_w_pool_matrix(w, k), dtype=jnp.bfloat16)

    lanes = max(w, 128)
    block_vmem = tb * k * h * lanes * 4
    vmem_need = 2 * block_vmem + 2 * tb * ho * max(wo, 128) * 4 \
        + tb * h * lanes * 8 + 2 * h * max(wo, 128) * 2
    vmem_limit = int(min(vmem_need + 4 * 1024 * 1024, 110 * 1024 * 1024))

    cost = pl.CostEstimate(
        flops=int(G * k * h * w + 2 * G * h * w * wo + G * h * wo),
        transcendentals=0,
        bytes_accessed=int(G * k * h * w * 4 + G * ho * wo * 4 + qw.size * 2))

    out = pl.pallas_call(
        functools.partial(_pool_kernel, k=k, inv_vol=1.0 / float(k ** 3)),
        out_shape=jax.ShapeDtypeStruct((G, ho, wo), x.dtype),
        grid=(num_i,),
        in_specs=[
            pl.BlockSpec((w, wo), lambda i: (0, 0)),
            pl.BlockSpec((tb, k, h, w), lambda i: (i, 0, 0, 0)),
        ],
        out_specs=pl.BlockSpec((tb, ho, wo), lambda i: (i, 0, 0)),
        compiler_params=pltpu.CompilerParams(
            dimension_semantics=("parallel",),
            vmem_limit_bytes=vmem_limit),
        cost_estimate=cost,
    )(qw, x4)

    return out.reshape(n, c, do, ho, wo)
